# Initial kernel scaffold; baseline (speedup 1.0000x reference)
import jax
import jax.numpy as jnp
from jax.experimental import pallas as pl


def kernel(x, edge_index, W1, b1, W2, b2, Wl1, bl1, Wl2, bl2):
    raise NotImplementedError("write your pallas kernel here")


# speedup vs baseline: 1.9546x; 1.9546x over previous
import functools

import jax
import jax.numpy as jnp
from jax import lax
from jax.experimental import pallas as pl
from jax.experimental.pallas import tpu as pltpu
from jax.experimental.pallas import tpu_sc as plsc

N = 10000
D = 256
HD = 128
E = 160000
NC = 2
NS = 16
NW = NC * NS
CH = 128
EW = 5120
EPAD = NW * EW
NCHUNK = EW // CH
NPAD = 10240
DUMP = N
RPT = NPAD // NS
BLK = 256
GRID = NPAD // BLK


def _make_agg(do_count: bool):
    mesh = plsc.VectorSubcoreMesh(core_axis_name="c", subcore_axis_name="s")

    out_type = [
        jax.ShapeDtypeStruct((NC, NPAD, HD), jnp.float32),
        jax.ShapeDtypeStruct((NC, NPAD, HD), jnp.float32),
    ]
    if do_count:
        out_type.append(jax.ShapeDtypeStruct((NC, NPAD, HD), jnp.float32))

    scratch = [
        pltpu.VMEM((CH,), jnp.int32),
        pltpu.VMEM((CH,), jnp.int32),
        pltpu.VMEM((CH, HD), jnp.float32),
        pltpu.VMEM_SHARED((NPAD, HD), jnp.float32),
        pltpu.SemaphoreType.DMA,
    ]

    def body(xh0, xh1, srcr, dstr, zer, *rest):
        if do_count:
            oner = rest[0]
            part0, part1, cntp = rest[1:4]
            src_v, dstm_v, rows_v, acc, sem = rest[4:]
        else:
            part0, part1 = rest[:2]
            src_v, dstm_v, rows_v, acc, sem = rest[2:]

        cid = lax.axis_index("c")
        sid = lax.axis_index("s")
        wid = cid * NS + sid
        ebase = wid * EW
        zbase = sid * RPT

        def one_pass(xh, part):
            pltpu.sync_copy(zer.at[pl.ds(zbase, RPT)], acc.at[pl.ds(zbase, RPT)])
            if xh is None:
                pltpu.sync_copy(oner, rows_v)
            plsc.subcore_barrier()

            def chunk(k, carry):
                off = ebase + k * CH
                pltpu.sync_copy(dstr.at[pl.ds(off, CH)], dstm_v)
                if xh is not None:
                    pltpu.sync_copy(srcr.at[pl.ds(off, CH)], src_v)
                    pltpu.async_copy(xh.at[src_v], rows_v, sem).wait()
                pltpu.sync_copy(rows_v, acc.at[dstm_v], add=True)
                return carry

            lax.fori_loop(0, NCHUNK, chunk, 0)
            plsc.subcore_barrier()
            pltpu.sync_copy(acc.at[pl.ds(zbase, RPT)],
                            part.at[cid, pl.ds(zbase, RPT)])
            plsc.subcore_barrier()

        one_pass(xh0, part0)
        one_pass(xh1, part1)
        if do_count:
            one_pass(None, cntp)

    return pl.kernel(body, out_type=tuple(out_type), mesh=mesh,
                     scratch_types=scratch)


_agg_l1 = _make_agg(True)
_agg_l2 = _make_agg(False)


def _conv1_body(p0_ref, p1_ref, cnt_ref, x_ref, w1_ref, b1_ref,
                h0_ref, h1_ref):
    x = x_ref[...]
    s0 = p0_ref[0] + p0_ref[1] + x[:, :HD]
    s1 = p1_ref[0] + p1_ref[1] + x[:, HD:]
    c = cnt_ref[0, :, 0:1] + cnt_ref[1, :, 0:1] + 1.0
    inv = 1.0 / c
    hp = jax.lax.Precision.HIGHEST
    h = (jnp.dot(s0 * inv, w1_ref[0:HD], precision=hp,
                 preferred_element_type=jnp.float32)
         + jnp.dot(s1 * inv, w1_ref[HD:D], precision=hp,
                   preferred_element_type=jnp.float32)
         + jnp.dot(x, w1_ref[D:2 * D], precision=hp,
                   preferred_element_type=jnp.float32)
         + b1_ref[...])
    h = jnp.maximum(h, 0.0)
    h0_ref[...] = h[:, :HD]
    h1_ref[...] = h[:, HD:]


def _conv2_head_body(q0_ref, q1_ref, cnt_ref, h0_ref, h1_ref,
                     w2_ref, b2_ref, wl1_ref, bl1_ref, wl2_ref, bl2_ref,
                     out_ref):
    h0 = h0_ref[...]
    h1 = h1_ref[...]
    s0 = q0_ref[0] + q0_ref[1] + h0
    s1 = q1_ref[0] + q1_ref[1] + h1
    c = cnt_ref[0, :, 0:1] + cnt_ref[1, :, 0:1] + 1.0
    inv = 1.0 / c
    hp = jax.lax.Precision.HIGHEST
    h2 = (jnp.dot(s0 * inv, w2_ref[0:HD], precision=hp,
                  preferred_element_type=jnp.float32)
          + jnp.dot(s1 * inv, w2_ref[HD:D], precision=hp,
                    preferred_element_type=jnp.float32)
          + jnp.dot(h0, w2_ref[D:D + HD], precision=hp,
                    preferred_element_type=jnp.float32)
          + jnp.dot(h1, w2_ref[D + HD:2 * D], precision=hp,
                    preferred_element_type=jnp.float32)
          + b2_ref[...])
    h2 = jnp.maximum(h2, 0.0)
    s = jnp.maximum(jnp.dot(h2, wl1_ref[...], precision=hp,
                            preferred_element_type=jnp.float32)
                    + bl1_ref[...], 0.0)
    out_ref[...] = (jnp.dot(s, wl2_ref[...], precision=hp,
                            preferred_element_type=jnp.float32)
                    + bl2_ref[...])


def _part_spec():
    return pl.BlockSpec((NC, BLK, HD), lambda i: (0, i, 0))


def _row_spec(w):
    return pl.BlockSpec((BLK, w), lambda i: (i, 0))


def _full_spec(shape):
    nd = len(shape)
    return pl.BlockSpec(shape, lambda i: (0,) * nd)


def kernel(x, edge_index, W1, b1, W2, b2, Wl1, bl1, Wl2, bl2):
    f32 = jnp.float32
    src = edge_index[0]
    dst = edge_index[1]
    dst = jnp.where(src == dst, DUMP, dst)
    padlen = EPAD - E
    src = jnp.concatenate([src, jnp.zeros((padlen,), jnp.int32)])
    dst = jnp.concatenate([dst, jnp.full((padlen,), DUMP, jnp.int32)])

    x0 = x[:, :HD]
    x1 = x[:, HD:]
    zer = jnp.zeros((NPAD, HD), f32)
    oner = jnp.ones((CH, HD), f32)
    xp = jnp.pad(x, ((0, NPAD - N), (0, 0)))

    b1r = b1.reshape(1, -1)
    b2r = b2.reshape(1, -1)
    bl1r = bl1.reshape(1, -1)
    wl2p = jnp.pad(Wl2, ((0, 0), (0, HD - Wl2.shape[1])))
    bl2p = jnp.pad(bl2, (0, HD - bl2.shape[0])).reshape(1, -1)

    p0, p1, cntp = _agg_l1(x0, x1, src, dst, zer, oner)

    h0, h1 = pl.pallas_call(
        _conv1_body,
        grid=(GRID,),
        in_specs=[
            _part_spec(), _part_spec(), _part_spec(),
            _row_spec(D),
            _full_spec(W1.shape), _full_spec((1, D)),
        ],
        out_specs=[_row_spec(HD), _row_spec(HD)],
        out_shape=[
            jax.ShapeDtypeStruct((NPAD, HD), f32),
            jax.ShapeDtypeStruct((NPAD, HD), f32),
        ],
    )(p0, p1, cntp, xp, W1, b1r)

    q0, q1 = _agg_l2(h0, h1, src, dst, zer)

    scores = pl.pallas_call(
        _conv2_head_body,
        grid=(GRID,),
        in_specs=[
            _part_spec(), _part_spec(), _part_spec(),
            _row_spec(HD), _row_spec(HD),
            _full_spec(W2.shape), _full_spec((1, D)),
            _full_spec(Wl1.shape), _full_spec((1, D)),
            _full_spec(wl2p.shape), _full_spec((1, HD)),
        ],
        out_specs=_row_spec(HD),
        out_shape=jax.ShapeDtypeStruct((NPAD, HD), f32),
    )(q0, q1, cntp, h0, h1, W2, b2r, Wl1, bl1r, wl2p, bl2p)

    return scores[:N, :Wl2.shape[1]]


# speedup vs baseline: 2.4372x; 1.2469x over previous
import functools

import jax
import jax.numpy as jnp
from jax import lax
from jax.experimental import pallas as pl
from jax.experimental.pallas import tpu as pltpu
from jax.experimental.pallas import tpu_sc as plsc

N = 10000
D = 256
HD = 128
E = 160000
NC = 2
NS = 16
NW = NC * NS
CH = 128
EW = 5120
EPAD = NW * EW
NCHUNK = EW // CH
NPAD = 10240
DUMP = N
RPT = NPAD // NS
BLK = 256
GRID = NPAD // BLK


def _make_agg(do_count: bool):
    mesh = plsc.VectorSubcoreMesh(core_axis_name="c", subcore_axis_name="s")

    out_type = [
        jax.ShapeDtypeStruct((NC, NPAD, HD), jnp.float32),
        jax.ShapeDtypeStruct((NC, NPAD, HD), jnp.float32),
    ]
    if do_count:
        out_type.append(jax.ShapeDtypeStruct((NC, NPAD, HD), jnp.float32))

    scratch = [
        pltpu.VMEM((2, CH), jnp.int32),
        pltpu.VMEM((2, CH), jnp.int32),
        pltpu.VMEM((2, CH, HD), jnp.float32),
        pltpu.VMEM_SHARED((NPAD, HD), jnp.float32),
        pltpu.SemaphoreType.DMA,
        pltpu.SemaphoreType.DMA,
        pltpu.SemaphoreType.DMA,
        pltpu.SemaphoreType.DMA,
    ]

    def body(xh0, xh1, srcr, dstr, zer, *rest):
        if do_count:
            oner = rest[0]
            part0, part1, cntp = rest[1:4]
            src_v, dstm_v, rows_v, acc, g0, g1, i0, i1 = rest[4:]
        else:
            part0, part1 = rest[:2]
            src_v, dstm_v, rows_v, acc, g0, g1, i0, i1 = rest[2:]
        gsem = (g0, g1)
        isem = (i0, i1)

        cid = lax.axis_index("c")
        sid = lax.axis_index("s")
        wid = cid * NS + sid
        ebase = wid * EW
        zbase = sid * RPT

        def gather_pass(xh, part):
            for b in range(2):
                off = ebase + b * CH
                pltpu.sync_copy(srcr.at[pl.ds(off, CH)], src_v.at[b])
                pltpu.sync_copy(dstr.at[pl.ds(off, CH)], dstm_v.at[b])
                pltpu.async_copy(xh.at[src_v.at[b]], rows_v.at[b], gsem[b])
            pltpu.sync_copy(zer.at[pl.ds(zbase, RPT)], acc.at[pl.ds(zbase, RPT)])
            plsc.subcore_barrier()

            def super_chunk(k2, carry):
                for b in range(2):
                    k = 2 * k2 + b
                    pltpu.make_async_copy(
                        xh.at[src_v.at[b]], rows_v.at[b], gsem[b]).wait()
                    pltpu.sync_copy(rows_v.at[b], acc.at[dstm_v.at[b]],
                                    add=True)
                    kn = jnp.minimum(k + 2, NCHUNK - 1)
                    offn = ebase + kn * CH
                    pltpu.sync_copy(srcr.at[pl.ds(offn, CH)], src_v.at[b])
                    pltpu.sync_copy(dstr.at[pl.ds(offn, CH)], dstm_v.at[b])
                    pltpu.async_copy(xh.at[src_v.at[b]], rows_v.at[b],
                                     gsem[b])
                return carry

            lax.fori_loop(0, NCHUNK // 2, super_chunk, 0)
            for b in range(2):
                pltpu.make_async_copy(
                    xh.at[src_v.at[b]], rows_v.at[b], gsem[b]).wait()
            plsc.subcore_barrier()
            pltpu.sync_copy(acc.at[pl.ds(zbase, RPT)],
                            part.at[cid, pl.ds(zbase, RPT)])
            plsc.subcore_barrier()

        def count_pass(part):
            pltpu.sync_copy(oner, rows_v.at[0])
            for b in range(2):
                off = ebase + b * CH
                pltpu.async_copy(dstr.at[pl.ds(off, CH)], dstm_v.at[b],
                                 isem[b])
            pltpu.sync_copy(zer.at[pl.ds(zbase, RPT)], acc.at[pl.ds(zbase, RPT)])
            plsc.subcore_barrier()

            def super_chunk(k2, carry):
                for b in range(2):
                    k = 2 * k2 + b
                    pltpu.make_async_copy(
                        dstr.at[pl.ds(ebase, CH)], dstm_v.at[b],
                        isem[b]).wait()
                    pltpu.sync_copy(rows_v.at[0], acc.at[dstm_v.at[b]],
                                    add=True)
                    kn = jnp.minimum(k + 2, NCHUNK - 1)
                    offn = ebase + kn * CH
                    pltpu.async_copy(dstr.at[pl.ds(offn, CH)], dstm_v.at[b],
                                     isem[b])
                return carry

            lax.fori_loop(0, NCHUNK // 2, super_chunk, 0)
            for b in range(2):
                pltpu.make_async_copy(dstr.at[pl.ds(ebase, CH)],
                                      dstm_v.at[b], isem[b]).wait()
            plsc.subcore_barrier()
            pltpu.sync_copy(acc.at[pl.ds(zbase, RPT)],
                            part.at[cid, pl.ds(zbase, RPT)])
            plsc.subcore_barrier()

        gather_pass(xh0, part0)
        gather_pass(xh1, part1)
        if do_count:
            count_pass(cntp)

    return pl.kernel(body, out_type=tuple(out_type), mesh=mesh,
                     scratch_types=scratch)


_agg_l1 = _make_agg(True)
_agg_l2 = _make_agg(False)


def _conv1_body(p0_ref, p1_ref, cnt_ref, x_ref, w1_ref, b1_ref,
                h0_ref, h1_ref):
    x = x_ref[...]
    s0 = p0_ref[0] + p0_ref[1] + x[:, :HD]
    s1 = p1_ref[0] + p1_ref[1] + x[:, HD:]
    c = cnt_ref[0, :, 0:1] + cnt_ref[1, :, 0:1] + 1.0
    inv = 1.0 / c
    hp = jax.lax.Precision.DEFAULT
    h = (jnp.dot(s0 * inv, w1_ref[0:HD], precision=hp,
                 preferred_element_type=jnp.float32)
         + jnp.dot(s1 * inv, w1_ref[HD:D], precision=hp,
                   preferred_element_type=jnp.float32)
         + jnp.dot(x, w1_ref[D:2 * D], precision=hp,
                   preferred_element_type=jnp.float32)
         + b1_ref[...])
    h = jnp.maximum(h, 0.0)
    h0_ref[...] = h[:, :HD]
    h1_ref[...] = h[:, HD:]


def _conv2_head_body(q0_ref, q1_ref, cnt_ref, h0_ref, h1_ref,
                     w2_ref, b2_ref, wl1_ref, bl1_ref, wl2_ref, bl2_ref,
                     out_ref):
    h0 = h0_ref[...]
    h1 = h1_ref[...]
    s0 = q0_ref[0] + q0_ref[1] + h0
    s1 = q1_ref[0] + q1_ref[1] + h1
    c = cnt_ref[0, :, 0:1] + cnt_ref[1, :, 0:1] + 1.0
    inv = 1.0 / c
    hp = jax.lax.Precision.DEFAULT
    h2 = (jnp.dot(s0 * inv, w2_ref[0:HD], precision=hp,
                  preferred_element_type=jnp.float32)
          + jnp.dot(s1 * inv, w2_ref[HD:D], precision=hp,
                    preferred_element_type=jnp.float32)
          + jnp.dot(h0, w2_ref[D:D + HD], precision=hp,
                    preferred_element_type=jnp.float32)
          + jnp.dot(h1, w2_ref[D + HD:2 * D], precision=hp,
                    preferred_element_type=jnp.float32)
          + b2_ref[...])
    h2 = jnp.maximum(h2, 0.0)
    s = jnp.maximum(jnp.dot(h2, wl1_ref[...], precision=hp,
                            preferred_element_type=jnp.float32)
                    + bl1_ref[...], 0.0)
    out_ref[...] = (jnp.dot(s, wl2_ref[...], precision=hp,
                            preferred_element_type=jnp.float32)
                    + bl2_ref[...])


def _part_spec():
    return pl.BlockSpec((NC, BLK, HD), lambda i: (0, i, 0))


def _row_spec(w):
    return pl.BlockSpec((BLK, w), lambda i: (i, 0))


def _full_spec(shape):
    nd = len(shape)
    return pl.BlockSpec(shape, lambda i: (0,) * nd)


def kernel(x, edge_index, W1, b1, W2, b2, Wl1, bl1, Wl2, bl2):
    f32 = jnp.float32
    src = edge_index[0]
    dst = edge_index[1]
    dst = jnp.where(src == dst, DUMP, dst)
    padlen = EPAD - E
    src = jnp.concatenate([src, jnp.zeros((padlen,), jnp.int32)])
    dst = jnp.concatenate([dst, jnp.full((padlen,), DUMP, jnp.int32)])

    x0 = x[:, :HD]
    x1 = x[:, HD:]
    zer = jnp.zeros((NPAD, HD), f32)
    oner = jnp.ones((CH, HD), f32)
    xp = jnp.pad(x, ((0, NPAD - N), (0, 0)))

    b1r = b1.reshape(1, -1)
    b2r = b2.reshape(1, -1)
    bl1r = bl1.reshape(1, -1)
    wl2p = jnp.pad(Wl2, ((0, 0), (0, HD - Wl2.shape[1])))
    bl2p = jnp.pad(bl2, (0, HD - bl2.shape[0])).reshape(1, -1)

    p0, p1, cntp = _agg_l1(x0, x1, src, dst, zer, oner)

    h0, h1 = pl.pallas_call(
        _conv1_body,
        grid=(GRID,),
        in_specs=[
            _part_spec(), _part_spec(), _part_spec(),
            _row_spec(D),
            _full_spec(W1.shape), _full_spec((1, D)),
        ],
        out_specs=[_row_spec(HD), _row_spec(HD)],
        out_shape=[
            jax.ShapeDtypeStruct((NPAD, HD), f32),
            jax.ShapeDtypeStruct((NPAD, HD), f32),
        ],
    )(p0, p1, cntp, xp, W1, b1r)

    q0, q1 = _agg_l2(h0, h1, src, dst, zer)

    scores = pl.pallas_call(
        _conv2_head_body,
        grid=(GRID,),
        in_specs=[
            _part_spec(), _part_spec(), _part_spec(),
            _row_spec(HD), _row_spec(HD),
            _full_spec(W2.shape), _full_spec((1, D)),
            _full_spec(Wl1.shape), _full_spec((1, D)),
            _full_spec(wl2p.shape), _full_spec((1, HD)),
        ],
        out_specs=_row_spec(HD),
        out_shape=jax.ShapeDtypeStruct((NPAD, HD), f32),
    )(q0, q1, cntp, h0, h1, W2, b2r, Wl1, bl1r, wl2p, bl2p)

    return scores[:N, :Wl2.shape[1]]
